# baseline (speedup 1.0000x reference)
import functools

import jax
import jax.numpy as jnp
from jax.experimental import pallas as pl
from jax.experimental.pallas import tpu as pltpu

_NUM_EMBEDDINGS = 256
_LATENT = 128
_BETA = 4.0
_ROWS_PER_BLOCK = 256
_TOPK = 3


def _vq_block(x_ref, e_ref, et_ref, out_ref, loss_ref):
    i = pl.program_id(0)
    x = x_ref[...]
    emb = e_ref[...]
    embT = et_ref[...]

    en = jnp.sum(emb * emb, axis=1)
    xe = jax.lax.dot_general(
        x, embT, (((1,), (0,)), ((), ())), preferred_element_type=jnp.float32,
        precision=jax.lax.Precision.HIGHEST)
    s = en[None, :] - 2.0 * xe

    iota = jax.lax.broadcasted_iota(jnp.int32, s.shape, 1)

    ui = jax.lax.bitcast_convert_type(s, jnp.int32)
    key = ui ^ jax.lax.shift_right_logical(
        jax.lax.shift_right_arithmetic(ui, 31), 1)
    packed = (key & jnp.int32(-256)) | iota

    cand_idx = []
    for _ in range(_TOPK):
        mk = jnp.min(packed, axis=1, keepdims=True)
        cand_idx.append(mk & jnp.int32(255))
        packed = jnp.where(packed == mk, jnp.int32(2147483647), packed)

    ohs = jnp.concatenate(
        [(iota == ci).astype(jnp.float32) for ci in cand_idx], axis=0)
    ek_all = jax.lax.dot_general(
        ohs, emb, (((1,), (0,)), ((), ())),
        preferred_element_type=jnp.float32,
        precision=jax.lax.Precision.HIGHEST)

    r = x.shape[0]
    best_d = None
    for k in range(_TOPK):
        ek = jax.lax.slice(ek_all, (k * r, 0), ((k + 1) * r, _LATENT))
        diff = x - ek
        dk = jnp.sqrt(jnp.sum(jnp.square(diff), axis=1, keepdims=True))
        if best_d is None:
            best_d, best_i, best_e = dk, cand_idx[k], ek
        else:
            take = (dk < best_d) | ((dk == best_d) & (cand_idx[k] < best_i))
            best_d = jnp.where(take, dk, best_d)
            best_i = jnp.where(take, cand_idx[k], best_i)
            best_e = jnp.where(take, ek, best_e)

    out_ref[...] = x + (best_e - x)

    part = jnp.sum(best_d * best_d).reshape(1, 1)

    @pl.when(i == 0)
    def _():
        loss_ref[...] = jnp.zeros((1, 1), jnp.float32)

    loss_ref[...] += part


@functools.partial(jax.jit, static_argnames=())
def kernel(inputs, embeddings):
    shape = inputs.shape
    n = shape[0] * shape[1] * shape[2]
    x = inputs.reshape(n, _LATENT)
    grid = n // _ROWS_PER_BLOCK

    out, loss_sum = pl.pallas_call(
        _vq_block,
        grid=(grid,),
        in_specs=[
            pl.BlockSpec((_ROWS_PER_BLOCK, _LATENT), lambda i: (i, 0)),
            pl.BlockSpec((_NUM_EMBEDDINGS, _LATENT), lambda i: (0, 0)),
            pl.BlockSpec((_LATENT, _NUM_EMBEDDINGS), lambda i: (0, 0)),
        ],
        out_specs=[
            pl.BlockSpec((_ROWS_PER_BLOCK, _LATENT), lambda i: (i, 0)),
            pl.BlockSpec((1, 1), lambda i: (0, 0)),
        ],
        out_shape=[
            jax.ShapeDtypeStruct((n, _LATENT), jnp.float32),
            jax.ShapeDtypeStruct((1, 1), jnp.float32),
        ],
        compiler_params=pltpu.CompilerParams(
            dimension_semantics=("arbitrary",),
        ),
    )(x, embeddings, embeddings.T)

    loss = loss_sum[0, 0] * ((1.0 + _BETA) / (n * _LATENT))
    return out.reshape(shape), loss


# speedup vs baseline: 17.4799x; 17.4799x over previous
import functools

import jax
import jax.numpy as jnp
from jax.experimental import pallas as pl
from jax.experimental.pallas import tpu as pltpu

_NUM_EMBEDDINGS = 256
_LATENT = 128
_BETA = 4.0
_ROWS_PER_BLOCK = 256
_TOPK = 3


def _vq_block(x_ref, e_ref, et_ref, out_ref, loss_ref):
    i = pl.program_id(0)
    x = x_ref[...]
    emb = e_ref[...]
    embT = et_ref[...]

    en = jnp.sum(embT * embT, axis=0, keepdims=True)
    xe = jax.lax.dot_general(
        x, embT, (((1,), (0,)), ((), ())), preferred_element_type=jnp.float32,
        precision=jax.lax.Precision.HIGHEST)
    s = en - 2.0 * xe

    iota = jax.lax.broadcasted_iota(jnp.int32, s.shape, 1)

    ui = jax.lax.bitcast_convert_type(s, jnp.int32)
    key = ui ^ jax.lax.shift_right_logical(
        jax.lax.shift_right_arithmetic(ui, 31), 1)
    packed = (key & jnp.int32(-256)) | iota

    cand_idx = []
    for _ in range(_TOPK):
        mk = jnp.min(packed, axis=1, keepdims=True)
        cand_idx.append(mk & jnp.int32(255))
        packed = jnp.where(packed == mk, jnp.int32(2147483647), packed)

    ohs = jnp.concatenate(
        [(iota == ci).astype(jnp.float32) for ci in cand_idx], axis=0)
    ek_all = jax.lax.dot_general(
        ohs, emb, (((1,), (0,)), ((), ())),
        preferred_element_type=jnp.float32,
        precision=jax.lax.Precision.HIGHEST)

    r = x.shape[0]
    best_d = None
    for k in range(_TOPK):
        ek = jax.lax.slice(ek_all, (k * r, 0), ((k + 1) * r, _LATENT))
        diff = x - ek
        dk = jnp.sqrt(jnp.sum(jnp.square(diff), axis=1, keepdims=True))
        if best_d is None:
            best_d, best_i, best_e = dk, cand_idx[k], ek
        else:
            take = (dk < best_d) | ((dk == best_d) & (cand_idx[k] < best_i))
            best_d = jnp.where(take, dk, best_d)
            best_i = jnp.where(take, cand_idx[k], best_i)
            best_e = jnp.where(take, ek, best_e)

    out_ref[...] = x + (best_e - x)

    part = jnp.sum(best_d * best_d).reshape(1, 1)

    @pl.when(i == 0)
    def _():
        loss_ref[...] = jnp.zeros((1, 1), jnp.float32)

    loss_ref[...] += part


@functools.partial(jax.jit, static_argnames=())
def kernel(inputs, embeddings):
    shape = inputs.shape
    n = shape[0] * shape[1] * shape[2]
    x = inputs.reshape(n, _LATENT)
    grid = n // _ROWS_PER_BLOCK

    out, loss_sum = pl.pallas_call(
        _vq_block,
        grid=(grid,),
        in_specs=[
            pl.BlockSpec((_ROWS_PER_BLOCK, _LATENT), lambda i: (i, 0)),
            pl.BlockSpec((_NUM_EMBEDDINGS, _LATENT), lambda i: (0, 0)),
            pl.BlockSpec((_LATENT, _NUM_EMBEDDINGS), lambda i: (0, 0)),
        ],
        out_specs=[
            pl.BlockSpec((_ROWS_PER_BLOCK, _LATENT), lambda i: (i, 0)),
            pl.BlockSpec((1, 1), lambda i: (0, 0)),
        ],
        out_shape=[
            jax.ShapeDtypeStruct((n, _LATENT), jnp.float32),
            jax.ShapeDtypeStruct((1, 1), jnp.float32),
        ],
        compiler_params=pltpu.CompilerParams(
            dimension_semantics=("arbitrary",),
        ),
    )(x, embeddings, embeddings.T)

    loss = loss_sum[0, 0] * ((1.0 + _BETA) / (n * _LATENT))
    return out.reshape(shape), loss


# speedup vs baseline: 22.9153x; 1.3110x over previous
import functools

import jax
import jax.numpy as jnp
from jax.experimental import pallas as pl
from jax.experimental.pallas import tpu as pltpu

_NUM_EMBEDDINGS = 256
_LATENT = 128
_BETA = 4.0
_ROWS_PER_BLOCK = 1024
_TOPK = 3


def _vq_block(x_ref, e_ref, et_ref, out_ref, loss_ref):
    i = pl.program_id(0)
    x = x_ref[...]
    emb = e_ref[...]
    embT = et_ref[...]

    en = jnp.sum(embT * embT, axis=0, keepdims=True)
    xe = jax.lax.dot_general(
        x, embT, (((1,), (0,)), ((), ())), preferred_element_type=jnp.float32,
        precision=jax.lax.Precision.HIGHEST)
    s = en - 2.0 * xe

    iota = jax.lax.broadcasted_iota(jnp.int32, s.shape, 1)

    ui = jax.lax.bitcast_convert_type(s, jnp.int32)
    key = ui ^ jax.lax.shift_right_logical(
        jax.lax.shift_right_arithmetic(ui, 31), 1)
    packed = (key & jnp.int32(-256)) | iota

    cand_idx = []
    for _ in range(_TOPK):
        mk = jnp.min(packed, axis=1, keepdims=True)
        cand_idx.append(mk & jnp.int32(255))
        packed = jnp.where(packed == mk, jnp.int32(2147483647), packed)

    ohs = jnp.concatenate(
        [(iota == ci).astype(jnp.float32) for ci in cand_idx], axis=0)
    ek_all = jax.lax.dot_general(
        ohs, emb, (((1,), (0,)), ((), ())),
        preferred_element_type=jnp.float32,
        precision=jax.lax.Precision.HIGHEST)

    r = x.shape[0]
    best_d = None
    for k in range(_TOPK):
        ek = jax.lax.slice(ek_all, (k * r, 0), ((k + 1) * r, _LATENT))
        diff = x - ek
        dk = jnp.sqrt(jnp.sum(jnp.square(diff), axis=1, keepdims=True))
        if best_d is None:
            best_d, best_i, best_e = dk, cand_idx[k], ek
        else:
            take = (dk < best_d) | ((dk == best_d) & (cand_idx[k] < best_i))
            best_d = jnp.where(take, dk, best_d)
            best_i = jnp.where(take, cand_idx[k], best_i)
            best_e = jnp.where(take, ek, best_e)

    out_ref[...] = x + (best_e - x)

    part = jnp.sum(best_d * best_d).reshape(1, 1)

    @pl.when(i == 0)
    def _():
        loss_ref[...] = jnp.zeros((1, 1), jnp.float32)

    loss_ref[...] += part


@functools.partial(jax.jit, static_argnames=())
def kernel(inputs, embeddings):
    shape = inputs.shape
    n = shape[0] * shape[1] * shape[2]
    x = inputs.reshape(n, _LATENT)
    grid = n // _ROWS_PER_BLOCK

    out, loss_sum = pl.pallas_call(
        _vq_block,
        grid=(grid,),
        in_specs=[
            pl.BlockSpec((_ROWS_PER_BLOCK, _LATENT), lambda i: (i, 0)),
            pl.BlockSpec((_NUM_EMBEDDINGS, _LATENT), lambda i: (0, 0)),
            pl.BlockSpec((_LATENT, _NUM_EMBEDDINGS), lambda i: (0, 0)),
        ],
        out_specs=[
            pl.BlockSpec((_ROWS_PER_BLOCK, _LATENT), lambda i: (i, 0)),
            pl.BlockSpec((1, 1), lambda i: (0, 0)),
        ],
        out_shape=[
            jax.ShapeDtypeStruct((n, _LATENT), jnp.float32),
            jax.ShapeDtypeStruct((1, 1), jnp.float32),
        ],
        compiler_params=pltpu.CompilerParams(
            dimension_semantics=("arbitrary",),
        ),
    )(x, embeddings, embeddings.T)

    loss = loss_sum[0, 0] * ((1.0 + _BETA) / (n * _LATENT))
    return out.reshape(shape), loss
